# baseline (speedup 1.0000x reference)
import functools

import jax
import jax.numpy as jnp
from jax import lax
from jax.experimental import pallas as pl
from jax.experimental.pallas import tpu as pltpu
from jax.experimental.pallas import tpu_sc as plsc

D = 64
L_POS = 512
NC = 2
NS = 16
LANES = 16
CHUNK = 128
NBUF = 4


@functools.lru_cache(maxsize=None)
def _build(N):
    NW = NC * NS
    per_w = N // NW
    nch = per_w // CHUNK
    ngrp = nch // NBUF
    phases = L_POS // CHUNK

    mesh = plsc.VectorSubcoreMesh(core_axis_name="c", subcore_axis_name="s")

    @functools.partial(
        pl.kernel,
        mesh=mesh,
        out_type=jax.ShapeDtypeStruct((N, D), jnp.float32),
        compiler_params=pltpu.CompilerParams(use_tc_tiling_on_sc=False),
        scratch_types=[pltpu.VMEM((per_w,), jnp.int32),
                       pltpu.VMEM((L_POS, D), jnp.float32)]
                      + [pltpu.VMEM((CHUNK, D), jnp.float32) for _ in range(NBUF)]
                      + [pltpu.SemaphoreType.DMA for _ in range(NBUF)],
    )
    def k(x_hbm, tok_hbm, pos_hbm, out_hbm, idx_v, pos_v, *rest):
        bufs = rest[:NBUF]
        sems = rest[NBUF:]
        wid = lax.axis_index("s") * NC + lax.axis_index("c")
        base = wid * per_w

        pltpu.sync_copy(x_hbm.at[pl.ds(base, per_w)], idx_v)
        pltpu.sync_copy(pos_hbm, pos_v)

        def gather_start(c, b):
            pltpu.async_copy(
                tok_hbm.at[idx_v.at[pl.ds(c * CHUNK, CHUNK)]], bufs[b], sems[b])

        def gather_wait(b):
            pltpu.make_async_copy(tok_hbm.at[pl.ds(0, CHUNK)], bufs[b], sems[b]).wait()

        def add_pos(b, c):
            pbase = (c % phases) * CHUNK
            def row(r, carry):
                for j in range(D // LANES):
                    plsc.addupdate(bufs[b].at[r, pl.ds(j * LANES, LANES)],
                                   pos_v[pbase + r, pl.ds(j * LANES, LANES)])
                return carry
            lax.fori_loop(0, CHUNK, row, 0)

        def do_chunk(g, b, start_next):
            c = g * NBUF + b
            gather_wait(b)
            add_pos(b, c)
            pltpu.sync_copy(bufs[b], out_hbm.at[pl.ds(base + c * CHUNK, CHUNK)])
            if start_next:
                gather_start(c + NBUF, b)

        for b in range(NBUF):
            gather_start(b, b)

        def group(g, carry):
            for b in range(NBUF):
                do_chunk(g, b, True)
            return carry
        lax.fori_loop(0, ngrp - 1, group, 0)
        for b in range(NBUF):
            do_chunk(ngrp - 1, b, False)

    return k


def kernel(x, token_table, pos_table):
    B, L = x.shape
    N = B * L
    xf = x.reshape(N).astype(jnp.int32)
    out = _build(N)(xf, token_table, pos_table)
    return out.reshape(B, L, D)
